# baseline (speedup 1.0000x reference)
import math

import jax
import jax.numpy as jnp
import numpy as np
from jax import lax
from jax.experimental import pallas as pl
from jax.experimental.pallas import tpu as pltpu
from jax.experimental.pallas import tpu_sc as plsc

D_EMB = 128
MAX_MODEL_LEN = 2048
B = 16
L = 2048

NC = 2
NS = 16
LANES = 16
NW = NC * NS
ROWS = B * L
CHUNK = 128
N_TOTAL_CHUNKS = ROWS // CHUNK
CH_PER_SEQ = L // CHUNK
CH_PER_W = N_TOTAL_CHUNKS // NW
HALF_W = CH_PER_W // 2
NBUF = 4
GROUPS = D_EMB // LANES
INV_SQRT_D = 1.0 / math.sqrt(D_EMB)


def _precompute_pe_t():
    pos_arg = np.arange(0, MAX_MODEL_LEN, dtype=np.float32)
    dim_arg = (10000.0 ** ((np.arange(0, D_EMB, dtype=np.float32) / 2.0)
                           / D_EMB)).reshape(-1, 1).astype(np.float32)
    pe = (pos_arg / dim_arg).astype(np.float32)
    pe[::2] = np.sin(pe[::2])
    pe[1::2] = np.cos(pe[1::2])
    return np.ascontiguousarray(pe.T)


_PE_T = _precompute_pe_t()
_ZEROS = np.zeros((CHUNK, D_EMB), np.float32)


def _tec_body(tokens_hbm, pe_hbm, len_hbm, emb_hbm, zeros_hbm, out_hbm,
              idx_v, rows0, rows1, rows2, rows3, pe_a, pe_b, len_v, z_sh,
              i_sem, pea_sem, peb_sem, z_sem,
              g_sem0, g_sem1, g_sem2, g_sem3,
              wb_sem0, wb_sem1, wb_sem2, wb_sem3):
    cid = lax.axis_index("c")
    sid = lax.axis_index("s")

    bs = [8 * cid + j for j in range(CH_PER_W)]
    pos_a = sid
    pos_b = (CH_PER_SEQ - 1) - sid
    ps = [pos_a] * HALF_W + [pos_b] * HALF_W
    cids = [bs[j] * CH_PER_SEQ + ps[j] for j in range(CH_PER_W)]

    idesc = []
    for j in range(CH_PER_W):
        d = pltpu.make_async_copy(
            tokens_hbm.at[pl.ds(cids[j], 1), :], idx_v.at[pl.ds(j, 1), :],
            i_sem)
        d.start()
        idesc.append(d)

    pea_d = pltpu.make_async_copy(
        pe_hbm.at[pl.ds(pos_a * CHUNK, CHUNK), :], pe_a, pea_sem)
    peb_d = pltpu.make_async_copy(
        pe_hbm.at[pl.ds(pos_b * CHUNK, CHUNK), :], pe_b, peb_sem)
    pea_d.start()
    peb_d.start()

    pltpu.sync_copy(len_hbm, len_v.at[pl.ds(0, B)])
    nvks = []
    for j in range(CH_PER_W):
        len_b = len_v[pl.ds(bs[j], LANES)][0]
        nvks.append(jnp.clip(len_b - ps[j] * CHUNK, 0, CHUNK))

    @pl.when(sid == 0)
    def _():
        pltpu.sync_copy(zeros_hbm, z_sh)

    plsc.subcore_barrier()

    bufs = [rows0, rows1, rows2, rows3]
    slabs = [pe_a] * HALF_W + [pe_b] * HALF_W
    g_sems = [g_sem0, g_sem1, g_sem2, g_sem3]
    wb_sems = [wb_sem0, wb_sem1, wb_sem2, wb_sem3]
    zero_vec = jnp.zeros((LANES,), jnp.float32)

    def g_desc(j):
        return pltpu.make_async_copy(
            emb_hbm.at[idx_v.at[j]], bufs[j % NBUF], g_sems[j % NBUF])

    def wb_desc(j):
        return pltpu.make_async_copy(
            bufs[j % NBUF],
            out_hbm.at[pl.ds(cids[j] * CHUNK, CHUNK), :], wb_sems[j % NBUF])

    def zwb_desc(j):
        return pltpu.make_async_copy(
            z_sh, out_hbm.at[pl.ds(cids[j] * CHUNK, CHUNK), :], z_sem)

    def issue_gather(j):
        @pl.when(nvks[j] > 0)
        def _():
            g_desc(j).start()

    def wait_gather(j):
        @pl.when(nvks[j] > 0)
        def _():
            g_desc(j).wait()

    def retire_wb(j):
        @pl.when(nvks[j] > 0)
        def _():
            wb_desc(j).wait()

    def compute_and_wb(j):
        nvk = nvks[j]
        buf = bufs[j % NBUF]
        slab = slabs[j]

        @pl.when(nvk > 0)
        def _():
            def body(r, carry):
                for c in range(GROUPS):
                    sl = pl.ds(c * LANES, LANES)
                    buf[r, sl] = (buf[r, sl] + slab[r, sl]) * INV_SQRT_D
                return carry

            lax.fori_loop(0, nvk, body, 0)

            def tail_body(r, carry):
                for c in range(GROUPS):
                    buf[r, pl.ds(c * LANES, LANES)] = zero_vec
                return carry

            lax.fori_loop(nvk, CHUNK, tail_body, 0)
            wb_desc(j).start()

        @pl.when(nvk <= 0)
        def _():
            zwb_desc(j).start()

    for d in idesc:
        d.wait()
    issue_gather(0)
    issue_gather(1)
    pea_d.wait()
    peb_d.wait()
    for j in range(CH_PER_W):
        wait_gather(j)
        if j + 2 < CH_PER_W:
            if j >= 2:
                retire_wb(j - 2)
            issue_gather(j + 2)
        compute_and_wb(j)
    for j in range(CH_PER_W - 4, CH_PER_W):
        if j >= 0:
            retire_wb(j)
    for j in range(CH_PER_W):
        @pl.when(nvks[j] <= 0)
        def _(j=j):
            zwb_desc(j).wait()


@jax.jit
def _run(tokens_2d, pe_t, lengths, emb_matrix, zeros):
    mesh = plsc.VectorSubcoreMesh(core_axis_name="c", subcore_axis_name="s",
                                  num_cores=NC, num_subcores=NS)
    out = pl.kernel(
        _tec_body,
        out_type=jax.ShapeDtypeStruct((ROWS, D_EMB), jnp.float32),
        mesh=mesh,
        scratch_types=[
            pltpu.VMEM((CH_PER_W, CHUNK), jnp.int32),
            pltpu.VMEM((CHUNK, D_EMB), jnp.float32),
            pltpu.VMEM((CHUNK, D_EMB), jnp.float32),
            pltpu.VMEM((CHUNK, D_EMB), jnp.float32),
            pltpu.VMEM((CHUNK, D_EMB), jnp.float32),
            pltpu.VMEM((CHUNK, D_EMB), jnp.float32),
            pltpu.VMEM((CHUNK, D_EMB), jnp.float32),
            pltpu.VMEM((B + LANES,), jnp.int32),
            pltpu.VMEM_SHARED((CHUNK, D_EMB), jnp.float32),
            pltpu.SemaphoreType.DMA,
            pltpu.SemaphoreType.DMA,
            pltpu.SemaphoreType.DMA,
            pltpu.SemaphoreType.DMA,
            pltpu.SemaphoreType.DMA,
            pltpu.SemaphoreType.DMA,
            pltpu.SemaphoreType.DMA,
            pltpu.SemaphoreType.DMA,
            pltpu.SemaphoreType.DMA,
            pltpu.SemaphoreType.DMA,
            pltpu.SemaphoreType.DMA,
            pltpu.SemaphoreType.DMA,
        ],
    )(tokens_2d, pe_t, lengths, emb_matrix, zeros)
    return out.reshape(B, L, D_EMB)


_DEV_CONSTS = {}


def kernel(tokens, lengths, emb_matrix):
    if "pe" not in _DEV_CONSTS:
        _DEV_CONSTS["pe"] = jax.device_put(_PE_T)
        _DEV_CONSTS["zeros"] = jax.device_put(_ZEROS)
    tokens_2d = tokens.reshape(N_TOTAL_CHUNKS, CHUNK).astype(jnp.int32)
    return _run(tokens_2d, _DEV_CONSTS["pe"],
                lengths.astype(jnp.int32), emb_matrix, _DEV_CONSTS["zeros"])


# speedup vs baseline: 1.5550x; 1.5550x over previous
import math

import jax
import jax.numpy as jnp
import numpy as np
from jax import lax
from jax.experimental import pallas as pl
from jax.experimental.pallas import tpu as pltpu
from jax.experimental.pallas import tpu_sc as plsc

D_EMB = 128
MAX_MODEL_LEN = 2048
B = 16
L = 2048

NC = 2
NS = 16
LANES = 16
NW = NC * NS
ROWS = B * L
CHUNK = 128
N_TOTAL_CHUNKS = ROWS // CHUNK
CH_PER_SEQ = L // CHUNK
CH_PER_W = N_TOTAL_CHUNKS // NW
HALF_W = CH_PER_W // 2
NBUF = 3
GROUPS = D_EMB // LANES
INV_SQRT_D = 1.0 / math.sqrt(D_EMB)


def _precompute_pe_t():
    pos_arg = np.arange(0, MAX_MODEL_LEN, dtype=np.float32)
    dim_arg = (10000.0 ** ((np.arange(0, D_EMB, dtype=np.float32) / 2.0)
                           / D_EMB)).reshape(-1, 1).astype(np.float32)
    pe = (pos_arg / dim_arg).astype(np.float32)
    pe[::2] = np.sin(pe[::2])
    pe[1::2] = np.cos(pe[1::2])
    return np.ascontiguousarray(pe.T)


_PE_T = _precompute_pe_t()
_ZEROS = np.zeros((CHUNK, D_EMB), np.float32)


def _tec_body(tokens_hbm, pe_hbm, len_hbm, emb_hbm, zeros_hbm, out_hbm,
              idx_v, rows0, rows1, rows2, len_v, pe_sh, z_sh,
              i_sem, pea_sem, z_sem, g_sem,
              pf_sem0, pf_sem1, pf_sem2, wb_sem0, wb_sem1, wb_sem2):
    cid = lax.axis_index("c")
    sid = lax.axis_index("s")

    wid = cid * NS + sid

    bs = [8 * cid + j for j in range(CH_PER_W)]
    ps = [lax.rem(wid + 2 * j, CH_PER_SEQ) for j in range(CH_PER_W)]
    cids = [bs[j] * CH_PER_SEQ + ps[j] for j in range(CH_PER_W)]

    idesc = []
    for j in range(CH_PER_W):
        d = pltpu.make_async_copy(
            tokens_hbm.at[pl.ds(cids[j], 1), :], idx_v.at[pl.ds(j, 1), :],
            i_sem)
        d.start()
        idesc.append(d)

    pe_stage = pltpu.make_async_copy(
        pe_hbm.at[pl.ds(sid * (L // NS), L // NS), :],
        pe_sh.at[pl.ds(sid * (L // NS), L // NS), :], pea_sem)
    pe_stage.start()

    pltpu.sync_copy(len_hbm, len_v.at[pl.ds(0, B)])
    nvks = []
    for j in range(CH_PER_W):
        len_b = len_v[pl.ds(bs[j], LANES)][0]
        nvks.append(jnp.clip(len_b - ps[j] * CHUNK, 0, CHUNK))

    @pl.when(sid == 0)
    def _():
        pltpu.sync_copy(zeros_hbm, z_sh)

    pe_stage.wait()
    plsc.subcore_barrier()

    bufs = [rows0, rows1, rows2]
    pf_sems = [pf_sem0, pf_sem1, pf_sem2]
    wb_sems = [wb_sem0, wb_sem1, wb_sem2]
    zero_vec = jnp.zeros((LANES,), jnp.float32)

    def pf_desc(j):
        return pltpu.make_async_copy(
            pe_sh.at[pl.ds(ps[j] * CHUNK, CHUNK), :],
            bufs[j % NBUF], pf_sems[j % NBUF])

    def g_desc(j):
        return pltpu.make_async_copy(
            emb_hbm.at[idx_v.at[j]], bufs[j % NBUF], g_sem)

    def wb_desc(j):
        return pltpu.make_async_copy(
            bufs[j % NBUF],
            out_hbm.at[pl.ds(cids[j] * CHUNK, CHUNK), :], wb_sems[j % NBUF])

    def zwb_desc(j):
        return pltpu.make_async_copy(
            z_sh, out_hbm.at[pl.ds(cids[j] * CHUNK, CHUNK), :], z_sem)

    def issue_pf(j):
        @pl.when(nvks[j] > 0)
        def _():
            pf_desc(j).start()

    def issue_gather(j):
        @pl.when(nvks[j] > 0)
        def _():
            pf_desc(j).wait()
            pltpu.async_copy(emb_hbm.at[idx_v.at[j]], bufs[j % NBUF], g_sem,
                             add=True)

    def wait_gather(j):
        @pl.when(nvks[j] > 0)
        def _():
            g_desc(j).wait()

    def retire_wb(j):
        @pl.when(nvks[j] > 0)
        def _():
            wb_desc(j).wait()

    def compute_and_wb(j):
        nvk = nvks[j]
        buf = bufs[j % NBUF]

        @pl.when(nvk > 0)
        def _():
            def scale_body(r, carry):
                for c in range(GROUPS):
                    sl = pl.ds(c * LANES, LANES)
                    buf[r, sl] = buf[r, sl] * INV_SQRT_D
                return carry

            lax.fori_loop(0, nvk, scale_body, 0)

            def tail_body(r, carry):
                for c in range(GROUPS):
                    buf[r, pl.ds(c * LANES, LANES)] = zero_vec
                return carry

            lax.fori_loop(nvk, CHUNK, tail_body, 0)
            wb_desc(j).start()

        @pl.when(nvk <= 0)
        def _():
            zwb_desc(j).start()

    for d in idesc:
        d.wait()
    issue_pf(0)
    issue_gather(0)
    issue_pf(1)
    for j in range(CH_PER_W):
        wait_gather(j)
        if j >= 1:
            retire_wb(j - 1)
        if j + 2 < CH_PER_W:
            issue_pf(j + 2)
        if j + 1 < CH_PER_W:
            issue_gather(j + 1)
        compute_and_wb(j)
    retire_wb(CH_PER_W - 1)
    for j in range(CH_PER_W):
        @pl.when(nvks[j] <= 0)
        def _(j=j):
            zwb_desc(j).wait()


@jax.jit
def _run(tokens_2d, pe_t, lengths, emb_matrix, zeros):
    mesh = plsc.VectorSubcoreMesh(core_axis_name="c", subcore_axis_name="s",
                                  num_cores=NC, num_subcores=NS)
    out = pl.kernel(
        _tec_body,
        out_type=jax.ShapeDtypeStruct((ROWS, D_EMB), jnp.float32),
        mesh=mesh,
        scratch_types=[
            pltpu.VMEM((CH_PER_W, CHUNK), jnp.int32),
            pltpu.VMEM((CHUNK, D_EMB), jnp.float32),
            pltpu.VMEM((CHUNK, D_EMB), jnp.float32),
            pltpu.VMEM((CHUNK, D_EMB), jnp.float32),
            pltpu.VMEM((B + LANES,), jnp.int32),
            pltpu.VMEM_SHARED((L, D_EMB), jnp.float32),
            pltpu.VMEM_SHARED((CHUNK, D_EMB), jnp.float32),
            pltpu.SemaphoreType.DMA,
            pltpu.SemaphoreType.DMA,
            pltpu.SemaphoreType.DMA,
            pltpu.SemaphoreType.DMA,
            pltpu.SemaphoreType.DMA,
            pltpu.SemaphoreType.DMA,
            pltpu.SemaphoreType.DMA,
            pltpu.SemaphoreType.DMA,
            pltpu.SemaphoreType.DMA,
            pltpu.SemaphoreType.DMA,
        ],
    )(tokens_2d, pe_t, lengths, emb_matrix, zeros)
    return out.reshape(B, L, D_EMB)


_DEV_CONSTS = {}


def kernel(tokens, lengths, emb_matrix):
    if "pe" not in _DEV_CONSTS:
        _DEV_CONSTS["pe"] = jax.device_put(_PE_T)
        _DEV_CONSTS["zeros"] = jax.device_put(_ZEROS)
    tokens_2d = tokens.reshape(N_TOTAL_CHUNKS, CHUNK).astype(jnp.int32)
    return _run(tokens_2d, _DEV_CONSTS["pe"],
                lengths.astype(jnp.int32), emb_matrix, _DEV_CONSTS["zeros"])


# speedup vs baseline: 1.5577x; 1.0018x over previous
import math

import jax
import jax.numpy as jnp
import numpy as np
from jax import lax
from jax.experimental import pallas as pl
from jax.experimental.pallas import tpu as pltpu
from jax.experimental.pallas import tpu_sc as plsc

D_EMB = 128
MAX_MODEL_LEN = 2048
B = 16
L = 2048

NC = 2
NS = 16
LANES = 16
NW = NC * NS
ROWS = B * L
CHUNK = 128
N_TOTAL_CHUNKS = ROWS // CHUNK
CH_PER_SEQ = L // CHUNK
CH_PER_W = N_TOTAL_CHUNKS // NW
NBUF = 3
GROUPS = D_EMB // LANES
INV_SQRT_D = 1.0 / math.sqrt(D_EMB)


def _precompute_pe_t():
    pos_arg = np.arange(0, MAX_MODEL_LEN, dtype=np.float32)
    dim_arg = (10000.0 ** ((np.arange(0, D_EMB, dtype=np.float32) / 2.0)
                           / D_EMB)).reshape(-1, 1).astype(np.float32)
    pe = (pos_arg / dim_arg).astype(np.float32)
    pe[::2] = np.sin(pe[::2])
    pe[1::2] = np.cos(pe[1::2])
    return np.ascontiguousarray(pe.T)


_PE_T = _precompute_pe_t()
_ZEROS = np.zeros((CHUNK, D_EMB), np.float32)


def _tec_body(tokens_hbm, pe_hbm, len_hbm, emb_hbm, zeros_hbm, out_hbm,
              idx_v, rows0, rows1, rows2, len_v, pe_sh, z_sh,
              i_sem, stage_sem, z_sem, g_sem,
              pf_sem0, pf_sem1, pf_sem2, wb_sem0, wb_sem1, wb_sem2):
    cid = lax.axis_index("c")
    sid = lax.axis_index("s")

    wid = cid * NS + sid

    bs = [8 * cid + j for j in range(CH_PER_W)]
    ps = [lax.rem(wid + 2 * j, CH_PER_SEQ) for j in range(CH_PER_W)]
    cids = [bs[j] * CH_PER_SEQ + ps[j] for j in range(CH_PER_W)]

    idesc = []
    for j in range(CH_PER_W):
        d = pltpu.make_async_copy(
            tokens_hbm.at[pl.ds(cids[j], 1), :], idx_v.at[pl.ds(j, 1), :],
            i_sem)
        d.start()
        idesc.append(d)

    pe_stage = pltpu.make_async_copy(
        pe_hbm.at[pl.ds(sid * (L // NS), L // NS), :],
        pe_sh.at[pl.ds(sid * (L // NS), L // NS), :], stage_sem)
    pe_stage.start()

    pltpu.sync_copy(len_hbm, len_v.at[pl.ds(0, B)])
    nvks = []
    for j in range(CH_PER_W):
        len_b = len_v[pl.ds(bs[j], LANES)][0]
        nvks.append(jnp.clip(len_b - ps[j] * CHUNK, 0, CHUNK))

    @pl.when(sid == 0)
    def _():
        pltpu.sync_copy(zeros_hbm, z_sh)

    pe_stage.wait()
    plsc.subcore_barrier()

    bufs = [rows0, rows1, rows2]
    pf_sems = [pf_sem0, pf_sem1, pf_sem2]
    wb_sems = [wb_sem0, wb_sem1, wb_sem2]
    zero_vec = jnp.zeros((LANES,), jnp.float32)

    def pf_desc(j):
        return pltpu.make_async_copy(
            pe_sh.at[pl.ds(ps[j] * CHUNK, CHUNK), :],
            bufs[j % NBUF], pf_sems[j % NBUF])

    def g_desc(j):
        return pltpu.make_async_copy(
            emb_hbm.at[idx_v.at[j]], bufs[j % NBUF], g_sem)

    def wb_desc(j):
        return pltpu.make_async_copy(
            bufs[j % NBUF],
            out_hbm.at[pl.ds(cids[j] * CHUNK, CHUNK), :], wb_sems[j % NBUF])

    def zwb_desc(j):
        return pltpu.make_async_copy(
            z_sh, out_hbm.at[pl.ds(cids[j] * CHUNK, CHUNK), :], z_sem)

    def issue_pf(j):
        @pl.when(nvks[j] > 0)
        def _():
            pf_desc(j).start()

    def issue_gather(j):
        @pl.when(nvks[j] > 0)
        def _():
            pf_desc(j).wait()
            pltpu.async_copy(emb_hbm.at[idx_v.at[j]], bufs[j % NBUF], g_sem,
                             add=True)

    def wait_gather(j):
        @pl.when(nvks[j] > 0)
        def _():
            g_desc(j).wait()

    def retire_wb(j):
        @pl.when(nvks[j] > 0)
        def _():
            wb_desc(j).wait()

    def compute_and_wb(j):
        nvk = nvks[j]
        buf = bufs[j % NBUF]

        @pl.when(nvk > 0)
        def _():
            def scale_body(r, carry):
                for c in range(GROUPS):
                    sl = pl.ds(c * LANES, LANES)
                    buf[r, sl] = buf[r, sl] * INV_SQRT_D
                return carry

            lax.fori_loop(0, nvk, scale_body, 0)

            def tail_body(r, carry):
                for c in range(GROUPS):
                    buf[r, pl.ds(c * LANES, LANES)] = zero_vec
                return carry

            lax.fori_loop(nvk, CHUNK, tail_body, 0)
            wb_desc(j).start()

        @pl.when(nvk <= 0)
        def _():
            zwb_desc(j).start()

    for d in idesc:
        d.wait()
    issue_pf(0)
    issue_gather(0)
    issue_pf(1)
    for j in range(CH_PER_W):
        wait_gather(j)
        if j >= 1:
            retire_wb(j - 1)
        if j + 2 < CH_PER_W:
            issue_pf(j + 2)
        if j + 1 < CH_PER_W:
            issue_gather(j + 1)
        compute_and_wb(j)
    retire_wb(CH_PER_W - 1)
    for j in range(CH_PER_W):
        @pl.when(nvks[j] <= 0)
        def _(j=j):
            zwb_desc(j).wait()


@jax.jit
def _run(tokens_2d, pe_t, lengths, emb_matrix, zeros):
    mesh = plsc.VectorSubcoreMesh(core_axis_name="c", subcore_axis_name="s",
                                  num_cores=NC, num_subcores=NS)
    out = pl.kernel(
        _tec_body,
        out_type=jax.ShapeDtypeStruct((ROWS, D_EMB), jnp.float32),
        mesh=mesh,
        scratch_types=[
            pltpu.VMEM((CH_PER_W, CHUNK), jnp.int32),
            pltpu.VMEM((CHUNK, D_EMB), jnp.float32),
            pltpu.VMEM((CHUNK, D_EMB), jnp.float32),
            pltpu.VMEM((CHUNK, D_EMB), jnp.float32),
            pltpu.VMEM((B + LANES,), jnp.int32),
            pltpu.VMEM_SHARED((L, D_EMB), jnp.float32),
            pltpu.VMEM_SHARED((CHUNK, D_EMB), jnp.float32),
            pltpu.SemaphoreType.DMA,
            pltpu.SemaphoreType.DMA,
            pltpu.SemaphoreType.DMA,
            pltpu.SemaphoreType.DMA,
            pltpu.SemaphoreType.DMA,
            pltpu.SemaphoreType.DMA,
            pltpu.SemaphoreType.DMA,
            pltpu.SemaphoreType.DMA,
            pltpu.SemaphoreType.DMA,
            pltpu.SemaphoreType.DMA,
        ],
    )(tokens_2d, pe_t, lengths, emb_matrix, zeros)
    return out.reshape(B, L, D_EMB)


_DEV_CONSTS = {}


def kernel(tokens, lengths, emb_matrix):
    if "pe" not in _DEV_CONSTS:
        _DEV_CONSTS["pe"] = jax.device_put(_PE_T)
        _DEV_CONSTS["zeros"] = jax.device_put(_ZEROS)
    tokens_2d = tokens.reshape(N_TOTAL_CHUNKS, CHUNK).astype(jnp.int32)
    return _run(tokens_2d, _DEV_CONSTS["pe"],
                lengths.astype(jnp.int32), emb_matrix, _DEV_CONSTS["zeros"])


# speedup vs baseline: 1.5784x; 1.0132x over previous
import math

import jax
import jax.numpy as jnp
import numpy as np
from jax import lax
from jax.experimental import pallas as pl
from jax.experimental.pallas import tpu as pltpu
from jax.experimental.pallas import tpu_sc as plsc

D_EMB = 128
MAX_MODEL_LEN = 2048
B = 16
L = 2048

NC = 2
NS = 16
LANES = 16
NW = NC * NS
ROWS = B * L
CHUNK = 128
N_TOTAL_CHUNKS = ROWS // CHUNK
CH_PER_SEQ = L // CHUNK
CH_PER_W = N_TOTAL_CHUNKS // NW
NBUF = 4
GROUPS = D_EMB // LANES
INV_SQRT_D = 1.0 / math.sqrt(D_EMB)


def _precompute_pe_t():
    pos_arg = np.arange(0, MAX_MODEL_LEN, dtype=np.float32)
    dim_arg = (10000.0 ** ((np.arange(0, D_EMB, dtype=np.float32) / 2.0)
                           / D_EMB)).reshape(-1, 1).astype(np.float32)
    pe = (pos_arg / dim_arg).astype(np.float32)
    pe[::2] = np.sin(pe[::2])
    pe[1::2] = np.cos(pe[1::2])
    return np.ascontiguousarray(pe.T)


_PE_T = _precompute_pe_t()
_ZEROS = np.zeros((CHUNK, D_EMB), np.float32)


def _tec_body(tokens_hbm, pe_hbm, len_hbm, emb_hbm, zeros_hbm, out_hbm,
              idx_v, rows0, rows1, rows2, rows3, len_v, pe_sh, z_sh,
              i_sem, stage_sem, z_sem, g_sem,
              pf_sem0, pf_sem1, pf_sem2, pf_sem3,
              wb_sem0, wb_sem1, wb_sem2, wb_sem3):
    cid = lax.axis_index("c")
    sid = lax.axis_index("s")

    wid = cid * NS + sid

    bs = [8 * cid + j for j in range(CH_PER_W)]
    ps = [lax.rem(wid + 2 * j, CH_PER_SEQ) for j in range(CH_PER_W)]
    cids = [bs[j] * CH_PER_SEQ + ps[j] for j in range(CH_PER_W)]

    idesc = []
    for j in range(CH_PER_W):
        d = pltpu.make_async_copy(
            tokens_hbm.at[pl.ds(cids[j], 1), :], idx_v.at[pl.ds(j, 1), :],
            i_sem)
        d.start()
        idesc.append(d)

    pe_stage = pltpu.make_async_copy(
        pe_hbm.at[pl.ds(sid * (L // NS), L // NS), :],
        pe_sh.at[pl.ds(sid * (L // NS), L // NS), :], stage_sem)
    pe_stage.start()

    pltpu.sync_copy(len_hbm, len_v.at[pl.ds(0, B)])
    nvks = []
    for j in range(CH_PER_W):
        len_b = len_v[pl.ds(bs[j], LANES)][0]
        nvks.append(jnp.clip(len_b - ps[j] * CHUNK, 0, CHUNK))

    @pl.when(sid == 0)
    def _():
        pltpu.sync_copy(zeros_hbm, z_sh)

    pe_stage.wait()
    plsc.subcore_barrier()

    bufs = [rows0, rows1, rows2, rows3]
    pf_sems = [pf_sem0, pf_sem1, pf_sem2, pf_sem3]
    wb_sems = [wb_sem0, wb_sem1, wb_sem2, wb_sem3]
    zero_vec = jnp.zeros((LANES,), jnp.float32)

    def pf_desc(j):
        return pltpu.make_async_copy(
            pe_sh.at[pl.ds(ps[j] * CHUNK, CHUNK), :],
            bufs[j % NBUF], pf_sems[j % NBUF])

    def g_desc(j):
        return pltpu.make_async_copy(
            emb_hbm.at[idx_v.at[j]], bufs[j % NBUF], g_sem)

    def wb_desc(j):
        return pltpu.make_async_copy(
            bufs[j % NBUF],
            out_hbm.at[pl.ds(cids[j] * CHUNK, CHUNK), :], wb_sems[j % NBUF])

    def zwb_desc(j):
        return pltpu.make_async_copy(
            z_sh, out_hbm.at[pl.ds(cids[j] * CHUNK, CHUNK), :], z_sem)

    def issue_pf(j):
        @pl.when(nvks[j] > 0)
        def _():
            pf_desc(j).start()

    def issue_gather(j):
        @pl.when(nvks[j] > 0)
        def _():
            pf_desc(j).wait()
            pltpu.async_copy(emb_hbm.at[idx_v.at[j]], bufs[j % NBUF], g_sem,
                             add=True)

    def wait_gather(j):
        @pl.when(nvks[j] > 0)
        def _():
            g_desc(j).wait()

    def retire_wb(j):
        @pl.when(nvks[j] > 0)
        def _():
            wb_desc(j).wait()

    def compute_and_wb(j):
        nvk = nvks[j]
        buf = bufs[j % NBUF]

        @pl.when(nvk > 0)
        def _():
            def scale_body(r, carry):
                for c in range(GROUPS):
                    sl = pl.ds(c * LANES, LANES)
                    buf[r, sl] = buf[r, sl] * INV_SQRT_D
                return carry

            lax.fori_loop(0, nvk, scale_body, 0)

            def tail_body(r, carry):
                for c in range(GROUPS):
                    buf[r, pl.ds(c * LANES, LANES)] = zero_vec
                return carry

            lax.fori_loop(nvk, CHUNK, tail_body, 0)
            wb_desc(j).start()

        @pl.when(nvk <= 0)
        def _():
            zwb_desc(j).start()

    for d in idesc:
        d.wait()
    issue_pf(0)
    issue_gather(0)
    issue_pf(1)
    for j in range(CH_PER_W):
        wait_gather(j)
        if j >= 2:
            retire_wb(j - 2)
        if j + 2 < CH_PER_W:
            issue_pf(j + 2)
        if j + 1 < CH_PER_W:
            issue_gather(j + 1)
        compute_and_wb(j)
    retire_wb(CH_PER_W - 2)
    retire_wb(CH_PER_W - 1)
    for j in range(CH_PER_W):
        @pl.when(nvks[j] <= 0)
        def _(j=j):
            zwb_desc(j).wait()


@jax.jit
def _run(tokens_2d, pe_t, lengths, emb_matrix, zeros):
    mesh = plsc.VectorSubcoreMesh(core_axis_name="c", subcore_axis_name="s",
                                  num_cores=NC, num_subcores=NS)
    out = pl.kernel(
        _tec_body,
        out_type=jax.ShapeDtypeStruct((ROWS, D_EMB), jnp.float32),
        mesh=mesh,
        scratch_types=[
            pltpu.VMEM((CH_PER_W, CHUNK), jnp.int32),
            pltpu.VMEM((CHUNK, D_EMB), jnp.float32),
            pltpu.VMEM((CHUNK, D_EMB), jnp.float32),
            pltpu.VMEM((CHUNK, D_EMB), jnp.float32),
            pltpu.VMEM((CHUNK, D_EMB), jnp.float32),
            pltpu.VMEM((B + LANES,), jnp.int32),
            pltpu.VMEM_SHARED((L, D_EMB), jnp.float32),
            pltpu.VMEM_SHARED((CHUNK, D_EMB), jnp.float32),
            pltpu.SemaphoreType.DMA,
            pltpu.SemaphoreType.DMA,
            pltpu.SemaphoreType.DMA,
            pltpu.SemaphoreType.DMA,
            pltpu.SemaphoreType.DMA,
            pltpu.SemaphoreType.DMA,
            pltpu.SemaphoreType.DMA,
            pltpu.SemaphoreType.DMA,
            pltpu.SemaphoreType.DMA,
            pltpu.SemaphoreType.DMA,
            pltpu.SemaphoreType.DMA,
            pltpu.SemaphoreType.DMA,
        ],
    )(tokens_2d, pe_t, lengths, emb_matrix, zeros)
    return out.reshape(B, L, D_EMB)


_DEV_CONSTS = {}


def kernel(tokens, lengths, emb_matrix):
    if "pe" not in _DEV_CONSTS:
        _DEV_CONSTS["pe"] = jax.device_put(_PE_T)
        _DEV_CONSTS["zeros"] = jax.device_put(_ZEROS)
    tokens_2d = tokens.reshape(N_TOTAL_CHUNKS, CHUNK).astype(jnp.int32)
    return _run(tokens_2d, _DEV_CONSTS["pe"],
                lengths.astype(jnp.int32), emb_matrix, _DEV_CONSTS["zeros"])
